# baseline (speedup 1.0000x reference)
import functools

import jax
import jax.numpy as jnp
import numpy as np
from jax import lax
from jax.experimental import pallas as pl
from jax.experimental.pallas import tpu as pltpu
from jax.experimental.pallas import tpu_sc as plsc

N_TGT_K = 10000
N_NBR_K = 100000
DEG_K = 32
E_K = N_TGT_K * DEG_K
D_K = 128
NV = D_K // 16

NW = 32
TPW = 320
EPW = TPW * DEG_K
PAD_T = NW * TPW
PAD_E = PAD_T * DEG_K

NSLICE = 64
SROWS = 1600
NLOAD = 10
SHARE = SROWS // NLOAD
BATCH = 128
WL_CAP = EPW + NSLICE * BATCH + BATCH
ACC_R = TPW + 8
TRASH = TPW << 11

_mesh = plsc.VectorSubcoreMesh(core_axis_name="c", subcore_axis_name="s")


@functools.partial(
    pl.kernel,
    out_type=jax.ShapeDtypeStruct((PAD_T, D_K), jnp.float32),
    mesh=_mesh,
    compiler_params=pltpu.CompilerParams(needs_layout_passes=False),
    scratch_types=[
        pltpu.VMEM((EPW,), jnp.int32),
        pltpu.VMEM((WL_CAP,), jnp.int32),
        pltpu.VMEM((1024,), jnp.int32),
        pltpu.SMEM((72,), jnp.int32),
        pltpu.VMEM((ACC_R, D_K), jnp.float32),
        pltpu.VMEM((BATCH, D_K), jnp.float32),
        pltpu.VMEM((BATCH, D_K), jnp.float32),
        pltpu.VMEM((BATCH,), jnp.int32),
        pltpu.VMEM((BATCH,), jnp.int32),
        pltpu.VMEM_SHARED((SROWS, D_K), jnp.float32),
        pltpu.VMEM_SHARED((SROWS, D_K), jnp.float32),
        pltpu.SemaphoreType.DMA,
        pltpu.SemaphoreType.DMA,
        pltpu.SemaphoreType.DMA,
        pltpu.SemaphoreType.DMA,
    ],
)
def _sc_gather_sum(col_hbm, table_hbm, out_hbm, idx_v, wl_v, cnt_v, bst_v,
                   acc_v, rb0_v, rb1_v, ib0_v, ib1_v, slA_v, slB_v,
                   ssemA, ssemB, g0, g1):
    cid = lax.axis_index("c")
    sid = lax.axis_index("s")
    wid = sid * 2 + cid

    lanes = lax.iota(jnp.int32, 16)
    ones = jnp.ones((16,), jnp.int32)
    zerov = jnp.zeros((16,), jnp.float32)

    pltpu.sync_copy(col_hbm.at[pl.ds(wid * EPW, EPW)], idx_v)

    def z1(i, _):
        cnt_v[pl.ds(i * 16, 16)] = jnp.zeros((16,), jnp.int32)
        return 0

    lax.fori_loop(0, 64, z1, 0)

    def z2(i, _):
        for d in range(NV):
            acc_v[i, pl.ds(d * 16, 16)] = zerov
        return 0

    lax.fori_loop(0, ACC_R, z2, 0)

    def z3(i, _):
        wl_v[pl.ds(i * 16, 16)] = jnp.full((16,), TRASH, jnp.int32)
        return 0

    lax.fori_loop(0, WL_CAP // 16, z3, 0)

    def hbody(i, _):
        colv = idx_v[pl.ds(i * 16, 16)]
        b = lax.div(colv, SROWS)
        bl = lax.shift_left(b, 4) + lanes
        old = plsc.load_gather(cnt_v, [bl])
        plsc.store_scatter(cnt_v, [bl], old + ones)
        return 0

    lax.fori_loop(0, EPW // 16, hbody, 0)

    def pbody(b, off):
        off = lax.bitwise_and(off + (BATCH - 1), ~(BATCH - 1))
        bst_v[b] = off
        row = cnt_v[pl.ds(b * 16, 16)]
        cs = plsc.cumsum(row)
        cnt_v[pl.ds(b * 16, 16)] = (cs - row) + lax.broadcast(off, (16,))
        return off + cs[15]

    off = lax.fori_loop(0, NSLICE, pbody, jnp.int32(0))
    bst_v[NSLICE] = lax.bitwise_and(off + (BATCH - 1), ~(BATCH - 1))

    def sbody(i, _):
        colv = idx_v[pl.ds(i * 16, 16)]
        b = lax.div(colv, SROWS)
        local = colv - b * SROWS
        local = local + lax.select(lax.eq(b, jnp.full((16,), 62, jnp.int32)),
                                   jnp.full((16,), 800, jnp.int32),
                                   jnp.zeros((16,), jnp.int32))
        tgt = lax.broadcast(i >> 1, (16,))
        word = lax.bitwise_or(lax.shift_left(tgt, 11), local)
        bl = lax.shift_left(b, 4) + lanes
        old = plsc.load_gather(cnt_v, [bl])
        plsc.store_scatter(cnt_v, [bl], old + 1)
        plsc.store_scatter(wl_v, [old], word)
        return 0

    lax.fori_loop(0, EPW // 16, sbody, 0)

    def sstart(p, buf, ssem):
        @pl.when(sid < NLOAD)
        def _():
            rbase = lax.min(p * SROWS, N_NBR_K - SROWS)
            pltpu.async_copy(
                table_hbm.at[pl.ds(rbase + sid * SHARE, SHARE)],
                buf.at[pl.ds(sid * SHARE, SHARE)], ssem)

    def swait(buf, ssem):
        @pl.when(sid < NLOAD)
        def _():
            pltpu.make_async_copy(
                table_hbm.at[pl.ds(0, SHARE)],
                buf.at[pl.ds(sid * SHARE, SHARE)], ssem).wait()

    sstart(0, slA_v, ssemA)
    sstart(1, slB_v, ssemB)

    def phase(p, buf, ssem):
        swait(buf, ssem)
        plsc.subcore_barrier()
        s0 = bst_v[p]
        s1 = bst_v[p + 1]
        nb = (s1 - s0) >> 7

        def prep(i, ib):
            base = s0 + i * BATCH
            for q in range(BATCH // 16):
                w = wl_v[pl.ds(base + q * 16, 16)]
                ib[pl.ds(q * 16, 16)] = lax.bitwise_and(w, 2047)

        def gstart(ib, rb, sem):
            pltpu.async_copy(buf.at[ib], rb, sem)

        def gwait(rb, sem):
            pltpu.make_async_copy(buf.at[ib0_v], rb, sem).wait()

        def accum_batch(i, rb):
            base = s0 + i * BATCH

            def mbody(m, _):
                tv = lax.shift_right_logical(wl_v[pl.ds(base + m * 16, 16)], 11)
                for j in range(16):
                    t = tv[j]
                    for d in range(NV):
                        plsc.addupdate(acc_v.at[t, pl.ds(d * 16, 16)],
                                       rb[m * 16 + j, pl.ds(d * 16, 16)])
                return 0

            lax.fori_loop(0, BATCH // 16, mbody, 0)

        @pl.when(nb > 0)
        def _():
            prep(0, ib0_v)
            gstart(ib0_v, rb0_v, g0)

        @pl.when(nb > 1)
        def _():
            prep(1, ib1_v)
            gstart(ib1_v, rb1_v, g1)

        def bloop(k, _):
            i0 = 2 * k
            i1 = i0 + 1
            gwait(rb0_v, g0)
            accum_batch(i0, rb0_v)

            @pl.when(i0 + 2 < nb)
            def _():
                prep(i0 + 2, ib0_v)
                gstart(ib0_v, rb0_v, g0)

            @pl.when(i1 < nb)
            def _():
                gwait(rb1_v, g1)
                accum_batch(i1, rb1_v)

                @pl.when(i1 + 2 < nb)
                def _():
                    prep(i1 + 2, ib1_v)
                    gstart(ib1_v, rb1_v, g1)

            return 0

        lax.fori_loop(0, (nb + 1) >> 1, bloop, 0)
        plsc.subcore_barrier()

        @pl.when(p + 2 < NSLICE)
        def _():
            sstart(p + 2, buf, ssem)

    def qbody(q, _):
        phase(2 * q, slA_v, ssemA)
        phase(2 * q + 1, slB_v, ssemB)
        return 0

    lax.fori_loop(0, NSLICE // 2, qbody, 0)

    pltpu.sync_copy(acc_v.at[pl.ds(0, TPW)], out_hbm.at[pl.ds(wid * TPW, TPW)])


def _mm_body(xt_ref, xs_ref, w1_ref, w2_ref, b_ref, o_ref):
    xs = xs_ref[...] * np.float32(1.0 / DEG_K)
    acc = jnp.dot(xt_ref[...], w1_ref[...], preferred_element_type=jnp.float32)
    acc = acc + jnp.dot(xs, w2_ref[...], preferred_element_type=jnp.float32)
    o_ref[...] = acc + b_ref[...]


_ROWS_BLK = 1000

_tc_linear = pl.pallas_call(
    _mm_body,
    grid=(N_TGT_K // _ROWS_BLK,),
    in_specs=[
        pl.BlockSpec((_ROWS_BLK, D_K), lambda i: (i, 0)),
        pl.BlockSpec((_ROWS_BLK, D_K), lambda i: (i, 0)),
        pl.BlockSpec((D_K, D_K), lambda i: (0, 0)),
        pl.BlockSpec((D_K, D_K), lambda i: (0, 0)),
        pl.BlockSpec((1, D_K), lambda i: (0, 0)),
    ],
    out_specs=pl.BlockSpec((_ROWS_BLK, D_K), lambda i: (i, 0)),
    out_shape=jax.ShapeDtypeStruct((N_TGT_K, D_K), jnp.float32),
)


def kernel(csr_row_ptr, csr_col_ind, sample_count, x_neighboor, x_target, W, b_lin, bias_param):
    col = csr_col_ind.astype(jnp.int32)
    col = jnp.concatenate([col, jnp.zeros((PAD_E - E_K,), jnp.int32)])
    xsum = _sc_gather_sum(col, x_neighboor)
    w1t = W[:, :D_K].T
    w2t = W[:, D_K:].T
    bvec = (b_lin + bias_param).reshape(1, D_K)
    return _tc_linear(x_target, xsum, w1t, w2t, bvec)


# speedup vs baseline: 2.0743x; 2.0743x over previous
import functools

import jax
import jax.numpy as jnp
import numpy as np
from jax import lax
from jax.experimental import pallas as pl
from jax.experimental.pallas import tpu as pltpu
from jax.experimental.pallas import tpu_sc as plsc

N_TGT_K = 10000
N_NBR_K = 100000
DEG_K = 32
E_K = N_TGT_K * DEG_K
D_K = 128

NW = 32
TPW = 320
CHUNK_T = 4
CHUNK_E = CHUNK_T * DEG_K
NCHUNK = TPW // CHUNK_T
PAD_T = NW * TPW
PAD_E = PAD_T * DEG_K

_mesh = plsc.VectorSubcoreMesh(core_axis_name="c", subcore_axis_name="s")


_NV = D_K // 16


@functools.partial(
    pl.kernel,
    out_type=jax.ShapeDtypeStruct((PAD_T, D_K), jnp.float32),
    mesh=_mesh,
    scratch_types=[
        pltpu.VMEM((NCHUNK, CHUNK_E), jnp.int32),
        pltpu.VMEM((CHUNK_E, D_K), jnp.float32),
        pltpu.VMEM((CHUNK_E, D_K), jnp.float32),
        pltpu.VMEM((CHUNK_E, D_K), jnp.float32),
        pltpu.VMEM((CHUNK_E, D_K), jnp.float32),
        pltpu.VMEM((TPW, D_K), jnp.float32),
        pltpu.SemaphoreType.DMA,
        pltpu.SemaphoreType.DMA,
        pltpu.SemaphoreType.DMA,
        pltpu.SemaphoreType.DMA,
    ],
)
def _sc_gather_sum(col_hbm, table_hbm, out_hbm, idx_v, rows0_v, rows1_v,
                   rows2_v, rows3_v, acc_v, sem0, sem1, sem2, sem3):
    wid = lax.axis_index("s") * 2 + lax.axis_index("c")
    pltpu.sync_copy(col_hbm.at[pl.ds(wid * NCHUNK, NCHUNK)], idx_v)

    def start(c, buf, sem):
        pltpu.async_copy(table_hbm.at[idx_v.at[c]], buf, sem)

    def wait(buf, sem):
        pltpu.make_async_copy(table_hbm.at[idx_v.at[0]], buf, sem).wait()

    def accum(buf, c):
        base = c * CHUNK_T
        for t in range(CHUNK_T):
            r0 = t * DEG_K
            init = tuple(buf[r0, pl.ds(d * 16, 16)] for d in range(_NV))

            def rbody(r, vs):
                return tuple(
                    vs[d] + buf[r0 + r, pl.ds(d * 16, 16)] for d in range(_NV)
                )

            vs = lax.fori_loop(1, DEG_K, rbody, init, unroll=4)
            for d in range(_NV):
                acc_v[base + t, pl.ds(d * 16, 16)] = vs[d]

    bufs = (rows0_v, rows1_v, rows2_v, rows3_v)
    sems = (sem0, sem1, sem2, sem3)
    NB = 4
    for k in range(NB):
        start(k, bufs[k], sems[k])

    def gbody(g, _):
        for k in range(NB):
            c = NB * g + k
            wait(bufs[k], sems[k])
            accum(bufs[k], c)

            @pl.when(g < NCHUNK // NB - 1)
            def _():
                start(c + NB, bufs[k], sems[k])

        return 0

    lax.fori_loop(0, NCHUNK // NB, gbody, 0)
    pltpu.sync_copy(acc_v, out_hbm.at[pl.ds(wid * TPW, TPW)])


def _mm_body(xt_ref, xs_ref, w1_ref, w2_ref, b_ref, o_ref):
    xs = xs_ref[...] * np.float32(1.0 / DEG_K)
    acc = jnp.dot(xt_ref[...], w1_ref[...], preferred_element_type=jnp.float32)
    acc = acc + jnp.dot(xs, w2_ref[...], preferred_element_type=jnp.float32)
    o_ref[...] = acc + b_ref[...]


_ROWS_BLK = 1000

_tc_linear = pl.pallas_call(
    _mm_body,
    grid=(N_TGT_K // _ROWS_BLK,),
    in_specs=[
        pl.BlockSpec((_ROWS_BLK, D_K), lambda i: (i, 0)),
        pl.BlockSpec((_ROWS_BLK, D_K), lambda i: (i, 0)),
        pl.BlockSpec((D_K, D_K), lambda i: (0, 0)),
        pl.BlockSpec((D_K, D_K), lambda i: (0, 0)),
        pl.BlockSpec((1, D_K), lambda i: (0, 0)),
    ],
    out_specs=pl.BlockSpec((_ROWS_BLK, D_K), lambda i: (i, 0)),
    out_shape=jax.ShapeDtypeStruct((N_TGT_K, D_K), jnp.float32),
)


def kernel(csr_row_ptr, csr_col_ind, sample_count, x_neighboor, x_target, W, b_lin, bias_param):
    col = csr_col_ind.astype(jnp.int32)
    col = jnp.concatenate([col, jnp.zeros((PAD_E - E_K,), jnp.int32)])
    col2d = col.reshape(NW * NCHUNK, CHUNK_E)
    xsum = _sc_gather_sum(col2d, x_neighboor)
    w1t = W[:, :D_K].T
    w2t = W[:, D_K:].T
    bvec = (b_lin + bias_param).reshape(1, D_K)
    return _tc_linear(x_target, xsum, w1t, w2t, bvec)
